# Initial kernel scaffold; baseline (speedup 1.0000x reference)
import functools

import jax
import jax.numpy as jnp
from jax import lax
from jax.experimental import pallas as pl
from jax.experimental.pallas import tpu as pltpu
from jax.experimental.pallas import tpu_sc as plsc

EMB = 32
B = 4096
L = 200
UPWEIGHT = 10.0
INV_DENOM = 1.0 / (L + UPWEIGHT - 1.0)

NC = 2
NS = 16
NW = NC * NS
BPW = B // NW
G0 = 128
G1 = L - G0
REDUCE_UNROLL = 8


def _sc_body(tokens_hbm, flagged_hbm, table_hbm, out_hbm,
             tokens_v, flagged_v, buf0, buf1, out_v, sem0, sem1):
    wid = lax.axis_index("s") * NC + lax.axis_index("c")
    base = wid * BPW

    pltpu.sync_copy(tokens_hbm.at[pl.ds(base, BPW)], tokens_v)
    pltpu.sync_copy(flagged_hbm.at[pl.ds(base, BPW)], flagged_v)

    def start_row(r, buf, sem):
        pltpu.async_copy(table_hbm.at[tokens_v.at[r, pl.ds(0, G0)]],
                         buf.at[pl.ds(0, G0)], sem)
        pltpu.async_copy(table_hbm.at[tokens_v.at[r, pl.ds(G0, G1)]],
                         buf.at[pl.ds(G0, G1)], sem)

    def wait_row(buf, sem):
        pltpu.make_async_copy(table_hbm.at[tokens_v.at[0, pl.ds(0, G0)]],
                              buf.at[pl.ds(0, G0)], sem).wait()
        pltpu.make_async_copy(table_hbm.at[tokens_v.at[0, pl.ds(G0, G1)]],
                              buf.at[pl.ds(G0, G1)], sem).wait()

    def reduce_row(r, buf):
        zero = jnp.zeros((16,), jnp.float32)

        def red(l, acc):
            a0, a1 = acc
            a0 = a0 + buf[l, pl.ds(0, 16)]
            a1 = a1 + buf[l, pl.ds(16, 16)]
            return (a0, a1)

        a0, a1 = lax.fori_loop(0, L, red, (zero, zero),
                               unroll=REDUCE_UNROLL)
        fl = flagged_v[r]
        e0 = buf[fl, pl.ds(0, 16)]
        e1 = buf[fl, pl.ds(16, 16)]
        a0 = (a0 + (UPWEIGHT - 1.0) * e0) * INV_DENOM
        a1 = (a1 + (UPWEIGHT - 1.0) * e1) * INV_DENOM
        out_v[r, pl.ds(0, 16)] = a0
        out_v[r, pl.ds(16, 16)] = a1

    start_row(0, buf0, sem0)

    def outer(i, carry):
        r = 2 * i
        start_row(r + 1, buf1, sem1)
        wait_row(buf0, sem0)
        reduce_row(r, buf0)

        @pl.when(r + 2 < BPW)
        def _():
            start_row(r + 2, buf0, sem0)

        wait_row(buf1, sem1)
        reduce_row(r + 1, buf1)
        return carry

    lax.fori_loop(0, BPW // 2, outer, 0)

    pltpu.sync_copy(out_v, out_hbm.at[pl.ds(base, BPW)])


def kernel(tokens, flagged_index, table):
    mesh = plsc.VectorSubcoreMesh(core_axis_name="c", subcore_axis_name="s")
    run = pl.kernel(
        _sc_body,
        out_type=jax.ShapeDtypeStruct((B, EMB), jnp.float32),
        mesh=mesh,
        scratch_types=[
            pltpu.VMEM((BPW, L), jnp.int32),
            pltpu.VMEM((BPW,), jnp.int32),
            pltpu.VMEM((L, EMB), jnp.float32),
            pltpu.VMEM((L, EMB), jnp.float32),
            pltpu.VMEM((BPW, EMB), jnp.float32),
            pltpu.SemaphoreType.DMA,
            pltpu.SemaphoreType.DMA,
        ],
    )
    return run(tokens, flagged_index, table)


# speedup vs baseline: 4.0391x; 4.0391x over previous
import functools

import jax
import jax.numpy as jnp
from jax import lax
from jax.experimental import pallas as pl
from jax.experimental.pallas import tpu as pltpu
from jax.experimental.pallas import tpu_sc as plsc

EMB = 32
B = 4096
L = 200
UPWEIGHT = 10.0
INV_DENOM = 1.0 / (L + UPWEIGHT - 1.0)

NC = 2
NS = 16
NW = NC * NS
BPW = B // NW
G0 = 128
G1 = L - G0
REDUCE_UNROLL = 8


def _sc_body(tokens_hbm, flagged_hbm, table_hbm, out_hbm,
             tokens_v, flagged_v, flag_tok_v, buf0, buf1, out_v,
             flag_rows_v, sem0, sem1, semf):
    wid = lax.axis_index("s") * NC + lax.axis_index("c")
    base = wid * BPW

    pltpu.sync_copy(tokens_hbm.at[pl.ds(base * L, BPW * L)], tokens_v)
    pltpu.sync_copy(flagged_hbm.at[pl.ds(base, BPW)], flagged_v)

    def start_row(r, buf, sem):
        pltpu.async_copy(table_hbm.at[tokens_v.at[pl.ds(r * L, G0)]],
                         buf.at[pl.ds(0, G0)], sem)
        pltpu.async_copy(table_hbm.at[tokens_v.at[pl.ds(r * L + G0, G1)]],
                         buf.at[pl.ds(G0, G1)], sem)

    def wait_row(buf, sem):
        pltpu.make_async_copy(table_hbm.at[tokens_v.at[pl.ds(0, G0)]],
                              buf.at[pl.ds(0, G0)], sem).wait()
        pltpu.make_async_copy(table_hbm.at[tokens_v.at[pl.ds(G0, G1)]],
                              buf.at[pl.ds(G0, G1)], sem).wait()

    def reduce_row(r, buf):
        zero = jnp.zeros((16,), jnp.float32)

        def red(l, acc):
            a0, a1 = acc
            a0 = a0 + buf[l, pl.ds(0, 16)]
            a1 = a1 + buf[l, pl.ds(16, 16)]
            return (a0, a1)

        a0, a1 = lax.fori_loop(0, L, red, (zero, zero),
                               unroll=REDUCE_UNROLL)
        out_v[r, pl.ds(0, 16)] = a0
        out_v[r, pl.ds(16, 16)] = a1

    lane = lax.iota(jnp.int32, 16)

    def flag_block(k, carry):
        rows16 = k * 16 + lane
        fl16 = flagged_v[pl.ds(k * 16, 16)]
        tok16 = plsc.load_gather(tokens_v, [rows16 * L + fl16])
        flag_tok_v[pl.ds(k * 16, 16)] = tok16
        return carry

    lax.fori_loop(0, BPW // 16, flag_block, 0)
    pltpu.async_copy(table_hbm.at[flag_tok_v], flag_rows_v, semf)

    start_row(0, buf0, sem0)

    def outer(i, carry):
        r = 2 * i
        start_row(r + 1, buf1, sem1)
        wait_row(buf0, sem0)
        reduce_row(r, buf0)

        @pl.when(r + 2 < BPW)
        def _():
            start_row(r + 2, buf0, sem0)

        wait_row(buf1, sem1)
        reduce_row(r + 1, buf1)
        return carry

    lax.fori_loop(0, BPW // 2, outer, 0)

    pltpu.make_async_copy(table_hbm.at[flag_tok_v], flag_rows_v, semf).wait()

    def fixup(r, carry):
        o0 = out_v[r, pl.ds(0, 16)]
        o1 = out_v[r, pl.ds(16, 16)]
        f0 = flag_rows_v[r, pl.ds(0, 16)]
        f1 = flag_rows_v[r, pl.ds(16, 16)]
        out_v[r, pl.ds(0, 16)] = (o0 + (UPWEIGHT - 1.0) * f0) * INV_DENOM
        out_v[r, pl.ds(16, 16)] = (o1 + (UPWEIGHT - 1.0) * f1) * INV_DENOM
        return carry

    lax.fori_loop(0, BPW, fixup, 0)

    pltpu.sync_copy(out_v, out_hbm.at[pl.ds(base, BPW)])


def kernel(tokens, flagged_index, table):
    mesh = plsc.VectorSubcoreMesh(core_axis_name="c", subcore_axis_name="s")
    run = pl.kernel(
        _sc_body,
        out_type=jax.ShapeDtypeStruct((B, EMB), jnp.float32),
        mesh=mesh,
        compiler_params=pltpu.CompilerParams(
            needs_layout_passes=False, use_tc_tiling_on_sc=False),
        scratch_types=[
            pltpu.VMEM((BPW * L,), jnp.int32),
            pltpu.VMEM((BPW,), jnp.int32),
            pltpu.VMEM((BPW,), jnp.int32),
            pltpu.VMEM((L, EMB), jnp.float32),
            pltpu.VMEM((L, EMB), jnp.float32),
            pltpu.VMEM((BPW, EMB), jnp.float32),
            pltpu.VMEM((BPW, EMB), jnp.float32),
            pltpu.SemaphoreType.DMA,
            pltpu.SemaphoreType.DMA,
            pltpu.SemaphoreType.DMA,
        ],
    )
    return run(tokens.reshape(-1), flagged_index, table)
